# baseline (speedup 1.0000x reference)
import dataclasses
import functools

import jax
import jax.numpy as jnp
from jax import lax
from jax.experimental import pallas as pl
from jax.experimental.pallas import tpu as pltpu
from jax.experimental.pallas import tpu_sc as plsc

N = 10000
E = 320000
D = 128
ED = 16
K = 32

NC = 2
NS = 16
NW = NC * NS

CE = 256
NCH = 40
PER_W = CE * NCH
TOT = NW * PER_W
NP = TOT // K

_mesh = plsc.VectorSubcoreMesh(core_axis_name="c", subcore_axis_name="s")

_no_layout = pltpu.CompilerParams()
if "needs_layout_passes" in pltpu.CompilerParams.__dataclass_fields__:
    _no_layout = dataclasses.replace(_no_layout, needs_layout_passes=False)


def _wid():
    return lax.axis_index("s") * NC + lax.axis_index("c")


@functools.partial(
    pl.kernel,
    out_type=jax.ShapeDtypeStruct((TOT, D), jnp.float32),
    mesh=_mesh,
    scratch_types=[
        pltpu.VMEM((CE,), jnp.int32),
        pltpu.VMEM((CE, D), jnp.float32),
    ],
)
def _sc_gather_x(x_hbm, src_hbm, xg_hbm, idx_v, rows_v):
    base0 = _wid() * PER_W

    @pl.loop(0, NCH)
    def _(i):
        base = base0 + i * CE
        pltpu.sync_copy(src_hbm.at[pl.ds(base, CE)], idx_v)
        pltpu.sync_copy(x_hbm.at[idx_v], rows_v)
        pltpu.sync_copy(rows_v, xg_hbm.at[pl.ds(base, CE)])


@functools.partial(
    pl.kernel,
    out_type=(
        jax.ShapeDtypeStruct((TOT, D), jnp.float32),
        jax.ShapeDtypeStruct((TOT,), jnp.int32),
    ),
    mesh=_mesh,
    scratch_types=[
        pltpu.VMEM((CE,), jnp.int32),
        pltpu.VMEM((CE, D), jnp.float32),
        pltpu.VMEM((CE,), jnp.int32),
        pltpu.VMEM((CE, D), jnp.int32),
        pltpu.VMEM((CE,), jnp.int32),
    ],
    compiler_params=_no_layout,
)
def _sc_gather_msg_dst(msg_hbm, nbr_hbm, dstrows_hbm, mg_hbm, dstg_hbm,
                       idx_v, rows_v, ridx_v, drows_v, didx_v):
    base0 = _wid() * PER_W

    @pl.loop(0, NCH)
    def _(i):
        base = base0 + i * CE
        pltpu.sync_copy(nbr_hbm.at[pl.ds(base, CE)], idx_v)
        pltpu.sync_copy(msg_hbm.at[idx_v], rows_v)
        pltpu.sync_copy(rows_v, mg_hbm.at[pl.ds(base, CE)])

        @pl.loop(0, CE // 16)
        def _(j):
            sl = pl.ds(j * 16, 16)
            ridx_v[sl] = lax.shift_right_logical(idx_v[sl], 7)

        pltpu.sync_copy(dstrows_hbm.at[ridx_v], drows_v)

        @pl.loop(0, CE // 16)
        def _(j):
            sl = pl.ds(j * 16, 16)
            r = lax.iota(jnp.int32, 16) + j * 16
            c = lax.bitwise_and(idx_v[sl], jnp.full((16,), 127, jnp.int32))
            didx_v[sl] = plsc.load_gather(drows_v, [r, c])

        pltpu.sync_copy(didx_v, dstg_hbm.at[pl.ds(base, CE)])


@functools.partial(
    pl.kernel,
    out_type=jax.ShapeDtypeStruct((TOT, D), jnp.float32),
    mesh=_mesh,
    scratch_types=[
        pltpu.VMEM((CE,), jnp.int32),
        pltpu.VMEM((CE, D), jnp.float32),
    ],
)
def _sc_gather_table(table_hbm, dstg_hbm, out_hbm, idx_v, rows_v):
    base0 = _wid() * PER_W

    @pl.loop(0, NCH)
    def _(i):
        base = base0 + i * CE
        pltpu.sync_copy(dstg_hbm.at[pl.ds(base, CE)], idx_v)
        pltpu.sync_copy(table_hbm.at[idx_v], rows_v)
        pltpu.sync_copy(rows_v, out_hbm.at[pl.ds(base, CE)])


_EB8 = 512


def _msg_body(xg_ref, ear_ref, wbig_ref, msg_ref):
    ea = jnp.dot(ear_ref[...], wbig_ref[...],
                 preferred_element_type=jnp.float32)
    msg_ref[...] = jax.nn.relu(xg_ref[...] + ea) + 1e-07


def _tc_msg(xg_r, ea_r, w_big):
    grid = (TOT // 8 // _EB8,)
    return pl.pallas_call(
        _msg_body,
        grid=grid,
        in_specs=[
            pl.BlockSpec((_EB8, 8 * D), lambda i: (i, 0)),
            pl.BlockSpec((_EB8, D), lambda i: (i, 0)),
            pl.BlockSpec((D, 8 * D), lambda i: (0, 0)),
        ],
        out_specs=pl.BlockSpec((_EB8, 8 * D), lambda i: (i, 0)),
        out_shape=jax.ShapeDtypeStruct((TOT // 8, 8 * D), jnp.float32),
    )(xg_r, ea_r, w_big)


_NB = 32


def _max_body(mg_ref, sm_ref):
    sm_ref[...] = jnp.max(mg_ref[...], axis=1)


def _tc_max(mg3):
    return pl.pallas_call(
        _max_body,
        grid=(NP // _NB,),
        in_specs=[pl.BlockSpec((_NB, K, D), lambda i: (i, 0, 0))],
        out_specs=pl.BlockSpec((_NB, D), lambda i: (i, 0)),
        out_shape=jax.ShapeDtypeStruct((NP, D), jnp.float32),
    )(mg3)


def _exp_body(mg_ref, smg_ref, os_ref, u_ref):
    t = jnp.exp(mg_ref[...] - smg_ref[...])
    u_ref[...] = mg_ref[...] * t
    os_ref[...] = jnp.sum(t, axis=1) + 1e-16


def _tc_expsum(mg3, smg3):
    return pl.pallas_call(
        _exp_body,
        grid=(NP // _NB,),
        in_specs=[
            pl.BlockSpec((_NB, K, D), lambda i: (i, 0, 0)),
            pl.BlockSpec((_NB, K, D), lambda i: (i, 0, 0)),
        ],
        out_specs=[
            pl.BlockSpec((_NB, D), lambda i: (i, 0)),
            pl.BlockSpec((_NB, K, D), lambda i: (i, 0, 0)),
        ],
        out_shape=[
            jax.ShapeDtypeStruct((NP, D), jnp.float32),
            jax.ShapeDtypeStruct((TOT // K, K, D), jnp.float32),
        ],
    )(mg3, smg3)


def _agg_body(u_ref, osg_ref, agg_ref):
    agg_ref[...] = jnp.sum(u_ref[...] / osg_ref[...], axis=1)


def _tc_agg(u3, osg3):
    return pl.pallas_call(
        _agg_body,
        grid=(NP // _NB,),
        in_specs=[
            pl.BlockSpec((_NB, K, D), lambda i: (i, 0, 0)),
            pl.BlockSpec((_NB, K, D), lambda i: (i, 0, 0)),
        ],
        out_specs=pl.BlockSpec((_NB, D), lambda i: (i, 0)),
        out_shape=jax.ShapeDtypeStruct((NP, D), jnp.float32),
    )(u3, osg3)


def _mlp_body(agg_ref, x_ref, w1t_ref, gamma_ref, beta_ref, w2t_ref, out_ref):
    out = agg_ref[...] + x_ref[...]
    h = jnp.dot(out, w1t_ref[...], preferred_element_type=jnp.float32)
    mean = jnp.mean(h, axis=0)
    var = jnp.mean((h - mean[None, :]) ** 2, axis=0)
    h = (h - mean[None, :]) / jnp.sqrt(var[None, :] + 1e-05)
    h = h * gamma_ref[...][None, :] + beta_ref[...][None, :]
    h = jax.nn.relu(h)
    out_ref[...] = jnp.dot(h, w2t_ref[...], preferred_element_type=jnp.float32)


def _tc_mlp(agg, x, w1t, gamma, beta, w2t):
    return pl.pallas_call(
        _mlp_body,
        out_shape=jax.ShapeDtypeStruct((N, D), jnp.float32),
    )(agg, x, w1t, gamma, beta, w2t)


def kernel(x, edge_index, edge_attr, nbr, W_edge, W1, gamma, beta, W2):
    src = edge_index[0]
    dst = edge_index[1]

    src_p = jnp.concatenate([src, jnp.zeros((TOT - E,), jnp.int32)])
    ea_p = jnp.concatenate(
        [edge_attr, jnp.zeros((TOT - E, ED), jnp.float32)], axis=0)
    ea_r = ea_p.reshape(TOT // 8, 8 * ED)
    dstrows = dst.reshape(E // D, D)
    nbr_p = jnp.concatenate(
        [nbr, jnp.zeros((NP - N, K), jnp.int32)], axis=0)
    nbrf = nbr_p.reshape(-1)

    wet = W_edge.T
    w_big = jax.scipy.linalg.block_diag(*([wet] * 8))
    w1t = W1.T
    w2t = W2.T

    xg = _sc_gather_x(x, src_p)
    xg_r = xg.reshape(TOT // 8, 8 * D)

    msg_r = _tc_msg(xg_r, ea_r, w_big)
    msg = msg_r.reshape(TOT, D)

    mg, dstg = _sc_gather_msg_dst(msg, nbrf, dstrows)
    mg3 = mg.reshape(NP, K, D)

    sm = _tc_max(mg3)
    smg = _sc_gather_table(sm, dstg)
    smg3 = smg.reshape(NP, K, D)

    osum, u = _tc_expsum(mg3, smg3)
    osg = _sc_gather_table(osum, dstg)
    osg3 = osg.reshape(NP, K, D)

    agg = _tc_agg(u.reshape(NP, K, D), osg3)

    return _tc_mlp(agg[:N], x, w1t, gamma, beta, w2t)


# speedup vs baseline: 1.1057x; 1.1057x over previous
import dataclasses
import functools

import jax
import jax.numpy as jnp
from jax import lax
from jax.experimental import pallas as pl
from jax.experimental.pallas import tpu as pltpu
from jax.experimental.pallas import tpu_sc as plsc

N = 10000
E = 320000
D = 128
ED = 16
K = 32

NC = 2
NS = 16
NW = NC * NS

PER_W = 10240
TOT = NW * PER_W
NP = TOT // K

CE = 320
NCH = PER_W // CE
CE3 = 160
NCH3 = PER_W // CE3

_mesh = plsc.VectorSubcoreMesh(core_axis_name="c", subcore_axis_name="s")

_no_layout = pltpu.CompilerParams()
if "needs_layout_passes" in pltpu.CompilerParams.__dataclass_fields__:
    _no_layout = dataclasses.replace(_no_layout, needs_layout_passes=False)


def _wid():
    return lax.axis_index("s") * NC + lax.axis_index("c")


def _make_row_gather(n_table_rows):
    @functools.partial(
        pl.kernel,
        out_type=jax.ShapeDtypeStruct((TOT, D), jnp.float32),
        mesh=_mesh,
        scratch_types=[
            pltpu.VMEM((PER_W,), jnp.int32),
            pltpu.VMEM((CE, D), jnp.float32),
            pltpu.VMEM((CE, D), jnp.float32),
            pltpu.SemaphoreType.DMA,
            pltpu.SemaphoreType.DMA,
            pltpu.SemaphoreType.DMA,
            pltpu.SemaphoreType.DMA,
        ],
    )
    def body(table_hbm, idx_hbm, out_hbm, idx_v, rows0, rows1,
             sg0, sg1, sw0, sw1):
        base0 = _wid() * PER_W
        pltpu.sync_copy(idx_hbm.at[pl.ds(base0, PER_W)], idx_v)
        rows = (rows0, rows1)
        sg = (sg0, sg1)
        sw = (sw0, sw1)

        def gather(c):
            b = c % 2
            return pltpu.make_async_copy(
                table_hbm.at[idx_v.at[pl.ds(c * CE, CE)]], rows[b], sg[b])

        def write(c):
            b = c % 2
            return pltpu.make_async_copy(
                rows[b], out_hbm.at[pl.ds(base0 + c * CE, CE)], sw[b])

        gather(0).start()
        for c in range(NCH):
            gather(c).wait()
            if c >= 1:
                write(c - 1).wait()
            if c + 1 < NCH:
                gather(c + 1).start()
            write(c).start()
        write(NCH - 1).wait()

    return body


_sc_gather_from_x = _make_row_gather(N)
_sc_gather_from_msg = _make_row_gather(TOT)
_sc_gather_from_table = _make_row_gather(NP)


@functools.partial(
    pl.kernel,
    out_type=jax.ShapeDtypeStruct((TOT,), jnp.int32),
    mesh=_mesh,
    scratch_types=[
        pltpu.VMEM((PER_W,), jnp.int32),
        pltpu.VMEM((PER_W,), jnp.int32),
        pltpu.VMEM((CE, D), jnp.int32),
        pltpu.VMEM((CE, D), jnp.int32),
        pltpu.VMEM((CE,), jnp.int32),
        pltpu.VMEM((CE,), jnp.int32),
        pltpu.SemaphoreType.DMA,
        pltpu.SemaphoreType.DMA,
        pltpu.SemaphoreType.DMA,
        pltpu.SemaphoreType.DMA,
    ],
    compiler_params=_no_layout,
)
def _sc_gather_dst(dstrows_hbm, nbr_hbm, dstg_hbm, idx_v, ridx_v,
                   drows0, drows1, didx0, didx1, sg0, sg1, sw0, sw1):
    base0 = _wid() * PER_W
    pltpu.sync_copy(nbr_hbm.at[pl.ds(base0, PER_W)], idx_v)

    @pl.loop(0, PER_W // 16)
    def _(j):
        sl = pl.ds(j * 16, 16)
        ridx_v[sl] = lax.shift_right_logical(idx_v[sl], 7)

    drows = (drows0, drows1)
    didx = (didx0, didx1)
    sg = (sg0, sg1)
    sw = (sw0, sw1)

    def gather(c):
        b = c % 2
        return pltpu.make_async_copy(
            dstrows_hbm.at[ridx_v.at[pl.ds(c * CE, CE)]], drows[b], sg[b])

    def write(c):
        b = c % 2
        return pltpu.make_async_copy(
            didx[b], dstg_hbm.at[pl.ds(base0 + c * CE, CE)], sw[b])

    gather(0).start()
    for c in range(NCH):
        b = c % 2
        gather(c).wait()
        if c >= 1:
            write(c - 1).wait()
        if c + 1 < NCH:
            gather(c + 1).start()

        @pl.loop(0, CE // 16)
        def _(j, b=b, c=c):
            sl = pl.ds(j * 16, 16)
            r = lax.iota(jnp.int32, 16) + j * 16
            col = lax.bitwise_and(idx_v[pl.ds(c * CE + j * 16, 16)],
                                  jnp.full((16,), 127, jnp.int32))
            didx[b][sl] = plsc.load_gather(drows[b], [r, col])

        write(c).start()
    write(NCH - 1).wait()


_EB8 = 512


def _msg_body(xg_ref, ear_ref, wbig_ref, msg_ref):
    ea = jnp.dot(ear_ref[...], wbig_ref[...],
                 preferred_element_type=jnp.float32)
    msg_ref[...] = jax.nn.relu(xg_ref[...] + ea) + 1e-07


def _tc_msg(xg_r, ea_r, w_big):
    grid = (TOT // 8 // _EB8,)
    return pl.pallas_call(
        _msg_body,
        grid=grid,
        in_specs=[
            pl.BlockSpec((_EB8, 8 * D), lambda i: (i, 0)),
            pl.BlockSpec((_EB8, D), lambda i: (i, 0)),
            pl.BlockSpec((D, 8 * D), lambda i: (0, 0)),
        ],
        out_specs=pl.BlockSpec((_EB8, 8 * D), lambda i: (i, 0)),
        out_shape=jax.ShapeDtypeStruct((TOT // 8, 8 * D), jnp.float32),
    )(xg_r, ea_r, w_big)


_NB = 32


def _max_body(mg_ref, sm_ref):
    sm_ref[...] = jnp.max(mg_ref[...], axis=1)


def _tc_max(mg3):
    return pl.pallas_call(
        _max_body,
        grid=(NP // _NB,),
        in_specs=[pl.BlockSpec((_NB, K, D), lambda i: (i, 0, 0))],
        out_specs=pl.BlockSpec((_NB, D), lambda i: (i, 0)),
        out_shape=jax.ShapeDtypeStruct((NP, D), jnp.float32),
    )(mg3)


def _exp_body(mg_ref, smg_ref, os_ref, u_ref):
    t = jnp.exp(mg_ref[...] - smg_ref[...])
    u_ref[...] = mg_ref[...] * t
    os_ref[...] = jnp.sum(t, axis=1) + 1e-16


def _tc_expsum(mg3, smg3):
    return pl.pallas_call(
        _exp_body,
        grid=(NP // _NB,),
        in_specs=[
            pl.BlockSpec((_NB, K, D), lambda i: (i, 0, 0)),
            pl.BlockSpec((_NB, K, D), lambda i: (i, 0, 0)),
        ],
        out_specs=[
            pl.BlockSpec((_NB, D), lambda i: (i, 0)),
            pl.BlockSpec((_NB, K, D), lambda i: (i, 0, 0)),
        ],
        out_shape=[
            jax.ShapeDtypeStruct((NP, D), jnp.float32),
            jax.ShapeDtypeStruct((TOT // K, K, D), jnp.float32),
        ],
    )(mg3, smg3)


def _agg_body(u_ref, osg_ref, agg_ref):
    agg_ref[...] = jnp.sum(u_ref[...] / osg_ref[...], axis=1)


def _tc_agg(u3, osg3):
    return pl.pallas_call(
        _agg_body,
        grid=(NP // _NB,),
        in_specs=[
            pl.BlockSpec((_NB, K, D), lambda i: (i, 0, 0)),
            pl.BlockSpec((_NB, K, D), lambda i: (i, 0, 0)),
        ],
        out_specs=pl.BlockSpec((_NB, D), lambda i: (i, 0)),
        out_shape=jax.ShapeDtypeStruct((NP, D), jnp.float32),
    )(u3, osg3)


def _mlp_body(agg_ref, x_ref, w1t_ref, gamma_ref, beta_ref, w2t_ref, out_ref):
    out = agg_ref[...] + x_ref[...]
    h = jnp.dot(out, w1t_ref[...], preferred_element_type=jnp.float32)
    mean = jnp.mean(h, axis=0)
    var = jnp.mean((h - mean[None, :]) ** 2, axis=0)
    h = (h - mean[None, :]) / jnp.sqrt(var[None, :] + 1e-05)
    h = h * gamma_ref[...][None, :] + beta_ref[...][None, :]
    h = jax.nn.relu(h)
    out_ref[...] = jnp.dot(h, w2t_ref[...], preferred_element_type=jnp.float32)


def _tc_mlp(agg, x, w1t, gamma, beta, w2t):
    return pl.pallas_call(
        _mlp_body,
        out_shape=jax.ShapeDtypeStruct((N, D), jnp.float32),
    )(agg, x, w1t, gamma, beta, w2t)


def kernel(x, edge_index, edge_attr, nbr, W_edge, W1, gamma, beta, W2):
    src = edge_index[0]
    dst = edge_index[1]

    src_p = jnp.concatenate([src, jnp.zeros((TOT - E,), jnp.int32)])
    ea_p = jnp.concatenate(
        [edge_attr, jnp.zeros((TOT - E, ED), jnp.float32)], axis=0)
    ea_r = ea_p.reshape(TOT // 8, 8 * ED)
    dstrows = dst.reshape(E // D, D)
    nbr_p = jnp.concatenate(
        [nbr, jnp.zeros((NP - N, K), jnp.int32)], axis=0)
    nbrf = nbr_p.reshape(-1)

    wet = W_edge.T
    w_big = jax.scipy.linalg.block_diag(*([wet] * 8))
    w1t = W1.T
    w2t = W2.T

    xg = _sc_gather_from_x(x, src_p)
    xg_r = xg.reshape(TOT // 8, 8 * D)

    dstg = _sc_gather_dst(dstrows, nbrf)

    msg_r = _tc_msg(xg_r, ea_r, w_big)
    msg = msg_r.reshape(TOT, D)

    mg = _sc_gather_from_msg(msg, nbrf)
    mg3 = mg.reshape(NP, K, D)

    sm = _tc_max(mg3)
    smg = _sc_gather_from_table(sm, dstg)
    smg3 = smg.reshape(NP, K, D)

    osum, u = _tc_expsum(mg3, smg3)
    osg = _sc_gather_from_table(osum, dstg)
    osg3 = osg.reshape(NP, K, D)

    agg = _tc_agg(u.reshape(NP, K, D), osg3)

    return _tc_mlp(agg[:N], x, w1t, gamma, beta, w2t)


# speedup vs baseline: 1.1906x; 1.0768x over previous
import dataclasses
import functools

import jax
import jax.numpy as jnp
from jax import lax
from jax.experimental import pallas as pl
from jax.experimental.pallas import tpu as pltpu
from jax.experimental.pallas import tpu_sc as plsc

N = 10000
E = 320000
D = 128
ED = 16
K = 32

NC = 2
NS = 16
NW = NC * NS

PER_W = 10240
TOT = NW * PER_W
NP = TOT // K

CE = 320

CH_C0 = 52
CH_C1 = 12
R0 = CH_C0 * CE
R1 = CH_C1 * CE
RP = R0 + R1

_mesh = plsc.VectorSubcoreMesh(core_axis_name="c", subcore_axis_name="s")

_no_layout = pltpu.CompilerParams()
if "needs_layout_passes" in pltpu.CompilerParams.__dataclass_fields__:
    _no_layout = dataclasses.replace(_no_layout, needs_layout_passes=False)


def _wid():
    return lax.axis_index("s") * NC + lax.axis_index("c")


def _make_row_gather(n_table_rows):
    @functools.partial(
        pl.kernel,
        out_type=jax.ShapeDtypeStruct((TOT, D), jnp.float32),
        mesh=_mesh,
        scratch_types=[
            pltpu.VMEM((R0,), jnp.int32),
            pltpu.VMEM((CE, D), jnp.float32),
            pltpu.VMEM((CE, D), jnp.float32),
            pltpu.SemaphoreType.DMA,
            pltpu.SemaphoreType.DMA,
            pltpu.SemaphoreType.DMA,
            pltpu.SemaphoreType.DMA,
        ],
    )
    def body(table_hbm, idx_hbm, out_hbm, idx_v, rows0, rows1,
             sg0, sg1, sw0, sw1):
        cid = lax.axis_index("c")
        pairbase = lax.axis_index("s") * RP
        rows = (rows0, rows1)
        sg = (sg0, sg1)
        sw = (sw0, sw1)

        def pipeline(base0, nch):
            pltpu.sync_copy(idx_hbm.at[pl.ds(base0, nch * CE)],
                            idx_v.at[pl.ds(0, nch * CE)])

            def gather(c):
                b = c % 2
                return pltpu.make_async_copy(
                    table_hbm.at[idx_v.at[pl.ds(c * CE, CE)]], rows[b], sg[b])

            def write(c):
                b = c % 2
                return pltpu.make_async_copy(
                    rows[b], out_hbm.at[pl.ds(base0 + c * CE, CE)], sw[b])

            gather(0).start()
            for c in range(nch):
                gather(c).wait()
                if c >= 1:
                    write(c - 1).wait()
                if c + 1 < nch:
                    gather(c + 1).start()
                write(c).start()
            write(nch - 1).wait()

        pl.when(cid == 0)(lambda: pipeline(pairbase, CH_C0))
        pl.when(cid != 0)(lambda: pipeline(pairbase + R0, CH_C1))

    return body


_sc_gather_from_x = _make_row_gather(N)
_sc_gather_from_msg = _make_row_gather(TOT)
_sc_gather_from_table = _make_row_gather(NP)


@functools.partial(
    pl.kernel,
    out_type=jax.ShapeDtypeStruct((TOT,), jnp.int32),
    mesh=_mesh,
    scratch_types=[
        pltpu.VMEM((R0,), jnp.int32),
        pltpu.VMEM((R0,), jnp.int32),
        pltpu.VMEM((CE, D), jnp.int32),
        pltpu.VMEM((CE, D), jnp.int32),
        pltpu.VMEM((CE,), jnp.int32),
        pltpu.VMEM((CE,), jnp.int32),
        pltpu.SemaphoreType.DMA,
        pltpu.SemaphoreType.DMA,
        pltpu.SemaphoreType.DMA,
        pltpu.SemaphoreType.DMA,
    ],
    compiler_params=_no_layout,
)
def _sc_gather_dst(dstrows_hbm, nbr_hbm, dstg_hbm, idx_v, ridx_v,
                   drows0, drows1, didx0, didx1, sg0, sg1, sw0, sw1):
    cid = lax.axis_index("c")
    pairbase = lax.axis_index("s") * RP
    drows = (drows0, drows1)
    didx = (didx0, didx1)
    sg = (sg0, sg1)
    sw = (sw0, sw1)

    def pipeline(base0, nch):
        pltpu.sync_copy(nbr_hbm.at[pl.ds(base0, nch * CE)],
                        idx_v.at[pl.ds(0, nch * CE)])

        @pl.loop(0, nch * CE // 16)
        def _(j):
            sl = pl.ds(j * 16, 16)
            ridx_v[sl] = lax.shift_right_logical(idx_v[sl], 7)

        def gather(c):
            b = c % 2
            return pltpu.make_async_copy(
                dstrows_hbm.at[ridx_v.at[pl.ds(c * CE, CE)]], drows[b], sg[b])

        def write(c):
            b = c % 2
            return pltpu.make_async_copy(
                didx[b], dstg_hbm.at[pl.ds(base0 + c * CE, CE)], sw[b])

        gather(0).start()
        for c in range(nch):
            b = c % 2
            gather(c).wait()
            if c >= 1:
                write(c - 1).wait()
            if c + 1 < nch:
                gather(c + 1).start()

            @pl.loop(0, CE // 16)
            def _(j, b=b, c=c):
                sl = pl.ds(j * 16, 16)
                r = lax.iota(jnp.int32, 16) + j * 16
                col = lax.bitwise_and(idx_v[pl.ds(c * CE + j * 16, 16)],
                                      jnp.full((16,), 127, jnp.int32))
                didx[b][sl] = plsc.load_gather(drows[b], [r, col])

            write(c).start()
        write(nch - 1).wait()

    pl.when(cid == 0)(lambda: pipeline(pairbase, CH_C0))
    pl.when(cid != 0)(lambda: pipeline(pairbase + R0, CH_C1))


_EB8 = 512


def _msg_body(xg_ref, ear_ref, wbig_ref, msg_ref):
    ea = jnp.dot(ear_ref[...], wbig_ref[...],
                 preferred_element_type=jnp.float32)
    msg_ref[...] = jax.nn.relu(xg_ref[...] + ea) + 1e-07


def _tc_msg(xg_r, ea_r, w_big):
    grid = (TOT // 8 // _EB8,)
    return pl.pallas_call(
        _msg_body,
        grid=grid,
        in_specs=[
            pl.BlockSpec((_EB8, 8 * D), lambda i: (i, 0)),
            pl.BlockSpec((_EB8, D), lambda i: (i, 0)),
            pl.BlockSpec((D, 8 * D), lambda i: (0, 0)),
        ],
        out_specs=pl.BlockSpec((_EB8, 8 * D), lambda i: (i, 0)),
        out_shape=jax.ShapeDtypeStruct((TOT // 8, 8 * D), jnp.float32),
    )(xg_r, ea_r, w_big)


_NB = 32


def _max_body(mg_ref, sm_ref):
    sm_ref[...] = jnp.max(mg_ref[...], axis=1)


def _tc_max(mg3):
    return pl.pallas_call(
        _max_body,
        grid=(NP // _NB,),
        in_specs=[pl.BlockSpec((_NB, K, D), lambda i: (i, 0, 0))],
        out_specs=pl.BlockSpec((_NB, D), lambda i: (i, 0)),
        out_shape=jax.ShapeDtypeStruct((NP, D), jnp.float32),
    )(mg3)


def _exp_body(mg_ref, smg_ref, os_ref, u_ref):
    t = jnp.exp(mg_ref[...] - smg_ref[...])
    u_ref[...] = mg_ref[...] * t
    os_ref[...] = jnp.sum(t, axis=1) + 1e-16


def _tc_expsum(mg3, smg3):
    return pl.pallas_call(
        _exp_body,
        grid=(NP // _NB,),
        in_specs=[
            pl.BlockSpec((_NB, K, D), lambda i: (i, 0, 0)),
            pl.BlockSpec((_NB, K, D), lambda i: (i, 0, 0)),
        ],
        out_specs=[
            pl.BlockSpec((_NB, D), lambda i: (i, 0)),
            pl.BlockSpec((_NB, K, D), lambda i: (i, 0, 0)),
        ],
        out_shape=[
            jax.ShapeDtypeStruct((NP, D), jnp.float32),
            jax.ShapeDtypeStruct((TOT // K, K, D), jnp.float32),
        ],
    )(mg3, smg3)


def _agg_body(u_ref, osg_ref, agg_ref):
    agg_ref[...] = jnp.sum(u_ref[...] / osg_ref[...], axis=1)


def _tc_agg(u3, osg3):
    return pl.pallas_call(
        _agg_body,
        grid=(NP // _NB,),
        in_specs=[
            pl.BlockSpec((_NB, K, D), lambda i: (i, 0, 0)),
            pl.BlockSpec((_NB, K, D), lambda i: (i, 0, 0)),
        ],
        out_specs=pl.BlockSpec((_NB, D), lambda i: (i, 0)),
        out_shape=jax.ShapeDtypeStruct((NP, D), jnp.float32),
    )(u3, osg3)


def _mlp_body(agg_ref, x_ref, w1t_ref, gamma_ref, beta_ref, w2t_ref, out_ref):
    out = agg_ref[...] + x_ref[...]
    h = jnp.dot(out, w1t_ref[...], preferred_element_type=jnp.float32)
    mean = jnp.mean(h, axis=0)
    var = jnp.mean((h - mean[None, :]) ** 2, axis=0)
    h = (h - mean[None, :]) / jnp.sqrt(var[None, :] + 1e-05)
    h = h * gamma_ref[...][None, :] + beta_ref[...][None, :]
    h = jax.nn.relu(h)
    out_ref[...] = jnp.dot(h, w2t_ref[...], preferred_element_type=jnp.float32)


def _tc_mlp(agg, x, w1t, gamma, beta, w2t):
    return pl.pallas_call(
        _mlp_body,
        out_shape=jax.ShapeDtypeStruct((N, D), jnp.float32),
    )(agg, x, w1t, gamma, beta, w2t)


def kernel(x, edge_index, edge_attr, nbr, W_edge, W1, gamma, beta, W2):
    src = edge_index[0]
    dst = edge_index[1]

    src_p = jnp.concatenate([src, jnp.zeros((TOT - E,), jnp.int32)])
    ea_p = jnp.concatenate(
        [edge_attr, jnp.zeros((TOT - E, ED), jnp.float32)], axis=0)
    ea_r = ea_p.reshape(TOT // 8, 8 * ED)
    dstrows = dst.reshape(E // D, D)
    nbr_p = jnp.concatenate(
        [nbr, jnp.zeros((NP - N, K), jnp.int32)], axis=0)
    nbrf = nbr_p.reshape(-1)

    wet = W_edge.T
    w_big = jax.scipy.linalg.block_diag(*([wet] * 8))
    w1t = W1.T
    w2t = W2.T

    xg = _sc_gather_from_x(x, src_p)
    xg_r = xg.reshape(TOT // 8, 8 * D)

    dstg = _sc_gather_dst(dstrows, nbrf)

    msg_r = _tc_msg(xg_r, ea_r, w_big)
    msg = msg_r.reshape(TOT, D)

    mg = _sc_gather_from_msg(msg, nbrf)
    mg3 = mg.reshape(NP, K, D)

    sm = _tc_max(mg3)
    smg = _sc_gather_from_table(sm, dstg)
    smg3 = smg.reshape(NP, K, D)

    osum, u = _tc_expsum(mg3, smg3)
    osg = _sc_gather_from_table(osum, dstg)
    osg3 = osg.reshape(NP, K, D)

    agg = _tc_agg(u.reshape(NP, K, D), osg3)

    return _tc_mlp(agg[:N], x, w1t, gamma, beta, w2t)
